# Initial kernel scaffold; baseline (speedup 1.0000x reference)
import jax
import jax.numpy as jnp
from jax.experimental import pallas as pl


def kernel(x, edge_index, W1, a_src1, a_dst1, b1, W2, a_src2, a_dst2, b2):
    raise NotImplementedError("write your pallas kernel here")


# speedup vs baseline: 58.4101x; 58.4101x over previous
import functools

import jax
import jax.numpy as jnp
from jax import lax
from jax.experimental import pallas as pl
from jax.experimental.pallas import tpu as pltpu
from jax.experimental.pallas import tpu_sc as plsc

N = 10000
NP = 10240
D_IN = 128
H = 16
C = 2
E_RAW = 320000
E_TOT = E_RAW + N
NC = 2
NS = 16
L = 16
NW = NC * NS
K = 1536
CHUNKS = 7
T = K * CHUNKS
E_PAD = T * NW
RPT = NP // NS
BN = 256


def _tc_a_body(x_ref, w_ref, av_ref, dv_ref,
               h_ref, as_ref, ad_ref, m_ref, mas_ref, mad_ref):
    i = pl.program_id(0)
    h = jnp.dot(x_ref[...], w_ref[...], preferred_element_type=jnp.float32)
    h_ref[...] = h
    a_s = jnp.sum(h * av_ref[...], axis=1, keepdims=True)
    a_d = jnp.sum(h * dv_ref[...], axis=1, keepdims=True)
    as_ref[...] = a_s
    ad_ref[...] = a_d
    bs = jnp.max(a_s)
    bd = jnp.max(a_d)

    @pl.when(i == 0)
    def _():
        mas_ref[0, 0] = bs
        mad_ref[0, 0] = bd

    @pl.when(i > 0)
    def _():
        mas_ref[0, 0] = jnp.maximum(mas_ref[0, 0], bs)
        mad_ref[0, 0] = jnp.maximum(mad_ref[0, 0], bd)

    @pl.when(i == pl.num_programs(0) - 1)
    def _():
        m_ref[0, 0] = jnp.maximum(mas_ref[0, 0] + mad_ref[0, 0], 0.0)


def _tc_a(xp, W1, a_src1, a_dst1):
    return pl.pallas_call(
        _tc_a_body,
        grid=(NP // BN,),
        in_specs=[
            pl.BlockSpec((BN, D_IN), lambda i: (i, 0)),
            pl.BlockSpec((D_IN, H), lambda i: (0, 0)),
            pl.BlockSpec((1, H), lambda i: (0, 0)),
            pl.BlockSpec((1, H), lambda i: (0, 0)),
        ],
        out_specs=[
            pl.BlockSpec((BN, H), lambda i: (i, 0)),
            pl.BlockSpec((BN, 1), lambda i: (i, 0)),
            pl.BlockSpec((BN, 1), lambda i: (i, 0)),
            pl.BlockSpec(memory_space=pltpu.SMEM),
        ],
        out_shape=[
            jax.ShapeDtypeStruct((NP, H), jnp.float32),
            jax.ShapeDtypeStruct((NP, 1), jnp.float32),
            jax.ShapeDtypeStruct((NP, 1), jnp.float32),
            jax.ShapeDtypeStruct((1, 1), jnp.float32),
        ],
        scratch_shapes=[
            pltpu.SMEM((1, 1), jnp.float32),
            pltpu.SMEM((1, 1), jnp.float32),
        ],
    )(xp, W1, a_src1.reshape(1, H), a_dst1.reshape(1, H))


def _sc1(srcp, dstp, as1, ad1, h1, mvec):
    mesh = plsc.VectorSubcoreMesh(core_axis_name="c", subcore_axis_name="s",
                                  num_cores=NC, num_subcores=NS)

    @functools.partial(
        pl.kernel,
        out_type=[
            jax.ShapeDtypeStruct((NC, NP, H), jnp.float32),
            jax.ShapeDtypeStruct((NC, NP), jnp.float32),
        ],
        mesh=mesh,
        compiler_params=pltpu.CompilerParams(needs_layout_passes=False, use_tc_tiling_on_sc=False),
        scratch_types=[
            pltpu.VMEM((NP,), jnp.float32),
            pltpu.VMEM((NP,), jnp.float32),
            pltpu.VMEM((K,), jnp.int32),
            pltpu.VMEM((K,), jnp.int32),
            pltpu.VMEM((K,), jnp.float32),
            pltpu.VMEM((K, H), jnp.float32),
            pltpu.VMEM((L,), jnp.float32),
            pltpu.VMEM_SHARED((NP, H), jnp.float32),
            pltpu.VMEM_SHARED((NP,), jnp.float32),
            pltpu.SemaphoreType.DMA,
        ],
    )
    def k(src_hbm, dst_hbm, as_hbm, ad_hbm, h_hbm, m_hbm, num_out, s_out,
          as_v, ad_v, src_v, dst_v, p_v, rows_v, m_v, num_sh, s_sh, sem):
        cid = lax.axis_index("c")
        sid = lax.axis_index("s")
        wid = cid * NS + sid

        pltpu.sync_copy(as_hbm, as_v)
        pltpu.sync_copy(ad_hbm, ad_v)
        pltpu.sync_copy(m_hbm, m_v)
        mvec_r = m_v[...]

        def zrow(i, _):
            rows_v[i, :] = jnp.zeros((H,), jnp.float32)
            return 0
        lax.fori_loop(0, RPT, zrow, 0)

        def zp(i, _):
            p_v[pl.ds(i * L, L)] = jnp.zeros((L,), jnp.float32)
            return 0
        lax.fori_loop(0, RPT // L, zp, 0)

        pltpu.sync_copy(rows_v.at[pl.ds(0, RPT)],
                        num_sh.at[pl.ds(sid * RPT, RPT)])
        pltpu.sync_copy(p_v.at[pl.ds(0, RPT)],
                        s_sh.at[pl.ds(sid * RPT, RPT)])
        plsc.subcore_barrier()

        base = wid * T

        def chunk(ci, _):
            off = base + ci * K
            pltpu.sync_copy(src_hbm.at[pl.ds(off, K)], src_v)
            pltpu.sync_copy(dst_hbm.at[pl.ds(off, K)], dst_v)
            pltpu.async_copy(h_hbm.at[src_v], rows_v, sem).wait()

            def grp(g, _):
                sidx = src_v[pl.ds(g * L, L)]
                didx = dst_v[pl.ds(g * L, L)]
                e = plsc.load_gather(as_v, [sidx]) + plsc.load_gather(ad_v, [didx])
                e = jnp.maximum(e, 0.2 * e)
                p = jnp.exp(e - mvec_r)
                gid = off + g * L + lax.iota(jnp.int32, L)
                p = jnp.where(gid < E_TOT, p, 0.0)
                p_v[pl.ds(g * L, L)] = p
                for j in range(L):
                    rows_v[g * L + j, :] = rows_v[g * L + j, :] * p[j]
                return 0
            lax.fori_loop(0, K // L, grp, 0)

            pltpu.sync_copy(rows_v, num_sh.at[dst_v], add=True)
            pltpu.sync_copy(p_v, s_sh.at[dst_v], add=True)
            return 0
        lax.fori_loop(0, CHUNKS, chunk, 0)

        plsc.subcore_barrier()
        pltpu.sync_copy(num_sh.at[pl.ds(sid * RPT, RPT)],
                        num_out.at[cid, pl.ds(sid * RPT, RPT)])
        pltpu.sync_copy(s_sh.at[pl.ds(sid * RPT, RPT)],
                        s_out.at[cid, pl.ds(sid * RPT, RPT)])

    return k(srcp, dstp, as1, ad1, h1, mvec)


def _tc_b_body(num_ref, s_ref, b1_ref, w2_ref, a2_ref,
               h0_ref, h1_ref, as2_ref, ad2_ref, m_ref, mas_ref, mad_ref):
    i = pl.program_id(0)
    num = num_ref[0] + num_ref[1]
    s = s_ref[0] + s_ref[1]
    s = jnp.where(s > 0.0, s, 1.0)
    out1 = num / s + b1_ref[...]
    h0 = jnp.sum(out1 * w2_ref[0:1, :], axis=1, keepdims=True)
    h1 = jnp.sum(out1 * w2_ref[1:2, :], axis=1, keepdims=True)
    h0_ref[...] = h0
    h1_ref[...] = h1
    a_s = h0 * a2_ref[0, 0] + h1 * a2_ref[0, 1]
    a_d = h0 * a2_ref[0, 2] + h1 * a2_ref[0, 3]
    as2_ref[...] = a_s
    ad2_ref[...] = a_d
    bs = jnp.max(a_s)
    bd = jnp.max(a_d)

    @pl.when(i == 0)
    def _():
        mas_ref[0, 0] = bs
        mad_ref[0, 0] = bd

    @pl.when(i > 0)
    def _():
        mas_ref[0, 0] = jnp.maximum(mas_ref[0, 0], bs)
        mad_ref[0, 0] = jnp.maximum(mad_ref[0, 0], bd)

    @pl.when(i == pl.num_programs(0) - 1)
    def _():
        m_ref[0, 0] = jnp.maximum(mas_ref[0, 0] + mad_ref[0, 0], 0.0)


def _tc_b(num_p, s_p, b1, W2, a_src2, a_dst2):
    w2t = W2.T.reshape(2, H)
    a2 = jnp.concatenate([a_src2, a_dst2]).reshape(1, 4)
    return pl.pallas_call(
        _tc_b_body,
        grid=(NP // BN,),
        in_specs=[
            pl.BlockSpec((NC, BN, H), lambda i: (0, i, 0)),
            pl.BlockSpec((NC, BN, 1), lambda i: (0, i, 0)),
            pl.BlockSpec((1, H), lambda i: (0, 0)),
            pl.BlockSpec((2, H), lambda i: (0, 0)),
            pl.BlockSpec(memory_space=pltpu.SMEM),
        ],
        out_specs=[
            pl.BlockSpec((BN, 1), lambda i: (i, 0)),
            pl.BlockSpec((BN, 1), lambda i: (i, 0)),
            pl.BlockSpec((BN, 1), lambda i: (i, 0)),
            pl.BlockSpec((BN, 1), lambda i: (i, 0)),
            pl.BlockSpec(memory_space=pltpu.SMEM),
        ],
        out_shape=[
            jax.ShapeDtypeStruct((NP, 1), jnp.float32),
            jax.ShapeDtypeStruct((NP, 1), jnp.float32),
            jax.ShapeDtypeStruct((NP, 1), jnp.float32),
            jax.ShapeDtypeStruct((NP, 1), jnp.float32),
            jax.ShapeDtypeStruct((1, 1), jnp.float32),
        ],
        scratch_shapes=[
            pltpu.SMEM((1, 1), jnp.float32),
            pltpu.SMEM((1, 1), jnp.float32),
        ],
    )(num_p, s_p.reshape(NC, NP, 1), b1.reshape(1, H), w2t, a2)


def _sc2(srcp, dstp, as2, ad2, h2c0, h2c1, mvec):
    mesh = plsc.VectorSubcoreMesh(core_axis_name="c", subcore_axis_name="s",
                                  num_cores=NC, num_subcores=NS)

    @functools.partial(
        pl.kernel,
        out_type=[
            jax.ShapeDtypeStruct((NC, NP), jnp.float32),
            jax.ShapeDtypeStruct((NC, NP), jnp.float32),
            jax.ShapeDtypeStruct((NC, NP), jnp.float32),
        ],
        mesh=mesh,
        compiler_params=pltpu.CompilerParams(needs_layout_passes=False, use_tc_tiling_on_sc=False),
        scratch_types=[
            pltpu.VMEM((NP,), jnp.float32),
            pltpu.VMEM((NP,), jnp.float32),
            pltpu.VMEM((NP,), jnp.float32),
            pltpu.VMEM((NP,), jnp.float32),
            pltpu.VMEM((K,), jnp.int32),
            pltpu.VMEM((K,), jnp.int32),
            pltpu.VMEM((K,), jnp.float32),
            pltpu.VMEM((K,), jnp.float32),
            pltpu.VMEM((K,), jnp.float32),
            pltpu.VMEM((L,), jnp.float32),
            pltpu.VMEM_SHARED((NP,), jnp.float32),
            pltpu.VMEM_SHARED((NP,), jnp.float32),
            pltpu.VMEM_SHARED((NP,), jnp.float32),
        ],
    )
    def k(src_hbm, dst_hbm, as_hbm, ad_hbm, c0_hbm, c1_hbm, m_hbm,
          n0_out, n1_out, s_out,
          as_v, ad_v, c0_v, c1_v, src_v, dst_v, p_v, v0_v, v1_v, m_v,
          n0_sh, n1_sh, s_sh):
        cid = lax.axis_index("c")
        sid = lax.axis_index("s")
        wid = cid * NS + sid

        pltpu.sync_copy(as_hbm, as_v)
        pltpu.sync_copy(ad_hbm, ad_v)
        pltpu.sync_copy(c0_hbm, c0_v)
        pltpu.sync_copy(c1_hbm, c1_v)
        pltpu.sync_copy(m_hbm, m_v)
        mvec_r = m_v[...]

        def zp(i, _):
            p_v[pl.ds(i * L, L)] = jnp.zeros((L,), jnp.float32)
            return 0
        lax.fori_loop(0, RPT // L, zp, 0)
        pltpu.sync_copy(p_v.at[pl.ds(0, RPT)], n0_sh.at[pl.ds(sid * RPT, RPT)])
        pltpu.sync_copy(p_v.at[pl.ds(0, RPT)], n1_sh.at[pl.ds(sid * RPT, RPT)])
        pltpu.sync_copy(p_v.at[pl.ds(0, RPT)], s_sh.at[pl.ds(sid * RPT, RPT)])
        plsc.subcore_barrier()

        base = wid * T

        def chunk(ci, _):
            off = base + ci * K
            pltpu.sync_copy(src_hbm.at[pl.ds(off, K)], src_v)
            pltpu.sync_copy(dst_hbm.at[pl.ds(off, K)], dst_v)

            def grp(g, _):
                sidx = src_v[pl.ds(g * L, L)]
                didx = dst_v[pl.ds(g * L, L)]
                e = plsc.load_gather(as_v, [sidx]) + plsc.load_gather(ad_v, [didx])
                e = jnp.maximum(e, 0.2 * e)
                p = jnp.exp(e - mvec_r)
                gid = off + g * L + lax.iota(jnp.int32, L)
                p = jnp.where(gid < E_TOT, p, 0.0)
                p_v[pl.ds(g * L, L)] = p
                v0_v[pl.ds(g * L, L)] = p * plsc.load_gather(c0_v, [sidx])
                v1_v[pl.ds(g * L, L)] = p * plsc.load_gather(c1_v, [sidx])
                return 0
            lax.fori_loop(0, K // L, grp, 0)

            pltpu.sync_copy(v0_v, n0_sh.at[dst_v], add=True)
            pltpu.sync_copy(v1_v, n1_sh.at[dst_v], add=True)
            pltpu.sync_copy(p_v, s_sh.at[dst_v], add=True)
            return 0
        lax.fori_loop(0, CHUNKS, chunk, 0)

        plsc.subcore_barrier()
        pltpu.sync_copy(n0_sh.at[pl.ds(sid * RPT, RPT)],
                        n0_out.at[cid, pl.ds(sid * RPT, RPT)])
        pltpu.sync_copy(n1_sh.at[pl.ds(sid * RPT, RPT)],
                        n1_out.at[cid, pl.ds(sid * RPT, RPT)])
        pltpu.sync_copy(s_sh.at[pl.ds(sid * RPT, RPT)],
                        s_out.at[cid, pl.ds(sid * RPT, RPT)])

    return k(srcp, dstp, as2, ad2, h2c0, h2c1, mvec)


def _tc_c_body(n0_ref, n1_ref, s_ref, b2_ref, out_ref, mn_ref, mx_ref):
    i = pl.program_id(0)
    n0 = n0_ref[0] + n0_ref[1]
    n1 = n1_ref[0] + n1_ref[1]
    s = s_ref[0] + s_ref[1]
    s = jnp.where(s > 0.0, s, 1.0)
    o = jnp.concatenate([n0 / s + b2_ref[0, 0], n1 / s + b2_ref[0, 1]], axis=1)
    out_ref[...] = o
    rid = i * BN + lax.broadcasted_iota(jnp.int32, (BN, C), 0)
    valid = rid < N
    bmn = jnp.min(jnp.where(valid, o, jnp.inf))
    bmx = jnp.max(jnp.where(valid, o, -jnp.inf))

    @pl.when(i == 0)
    def _():
        mn_ref[0, 0] = bmn
        mx_ref[0, 0] = bmx

    @pl.when(i > 0)
    def _():
        mn_ref[0, 0] = jnp.minimum(mn_ref[0, 0], bmn)
        mx_ref[0, 0] = jnp.maximum(mx_ref[0, 0], bmx)


def _tc_c(n0_p, n1_p, s2_p, b2):
    return pl.pallas_call(
        _tc_c_body,
        grid=(NP // BN,),
        in_specs=[
            pl.BlockSpec((NC, BN, 1), lambda i: (0, i, 0)),
            pl.BlockSpec((NC, BN, 1), lambda i: (0, i, 0)),
            pl.BlockSpec((NC, BN, 1), lambda i: (0, i, 0)),
            pl.BlockSpec(memory_space=pltpu.SMEM),
        ],
        out_specs=[
            pl.BlockSpec((BN, C), lambda i: (i, 0)),
            pl.BlockSpec(memory_space=pltpu.SMEM),
            pl.BlockSpec(memory_space=pltpu.SMEM),
        ],
        out_shape=[
            jax.ShapeDtypeStruct((NP, C), jnp.float32),
            jax.ShapeDtypeStruct((1, 1), jnp.float32),
            jax.ShapeDtypeStruct((1, 1), jnp.float32),
        ],
    )(n0_p.reshape(NC, NP, 1), n1_p.reshape(NC, NP, 1),
      s2_p.reshape(NC, NP, 1), b2.reshape(1, 2))


def _tc_d_body(o_ref, mn_ref, mx_ref, y_ref):
    mn = mn_ref[0, 0]
    mx = mx_ref[0, 0]
    y_ref[...] = 2.0 * ((o_ref[...] - mn) / (mx - mn)) - 1.0


def _tc_d(out2, mn, mx):
    return pl.pallas_call(
        _tc_d_body,
        grid=(NP // BN,),
        in_specs=[
            pl.BlockSpec((BN, C), lambda i: (i, 0)),
            pl.BlockSpec(memory_space=pltpu.SMEM),
            pl.BlockSpec(memory_space=pltpu.SMEM),
        ],
        out_specs=pl.BlockSpec((BN, C), lambda i: (i, 0)),
        out_shape=jax.ShapeDtypeStruct((NP, C), jnp.float32),
    )(out2, mn, mx)


def kernel(x, edge_index, W1, a_src1, a_dst1, b1, W2, a_src2, a_dst2, b2):
    xp = jnp.zeros((NP, D_IN), jnp.float32).at[:N].set(x)
    loop = jnp.arange(N, dtype=jnp.int32)
    pad = jnp.arange(E_PAD - E_TOT, dtype=jnp.int32) % N
    srcp = jnp.concatenate([edge_index[0].astype(jnp.int32), loop, pad])
    dstp = jnp.concatenate([edge_index[1].astype(jnp.int32), loop, pad])

    h1, as1, ad1, m1 = _tc_a(xp, W1, a_src1, a_dst1)
    mvec1 = jnp.broadcast_to(m1.reshape(1), (L,))
    num_p, s_p = _sc1(srcp, dstp, as1.reshape(NP), ad1.reshape(NP), h1, mvec1)

    h2c0, h2c1, as2, ad2, m2 = _tc_b(num_p, s_p, b1, W2, a_src2, a_dst2)
    mvec2 = jnp.broadcast_to(m2.reshape(1), (L,))
    n0_p, n1_p, s2_p = _sc2(srcp, dstp, as2.reshape(NP), ad2.reshape(NP),
                            h2c0.reshape(NP), h2c1.reshape(NP), mvec2)

    out2, mn, mx = _tc_c(n0_p, n1_p, s2_p, b2)
    y = _tc_d(out2, mn, mx)
    return y[:N]


# speedup vs baseline: 72.2328x; 1.2366x over previous
import functools

import jax
import jax.numpy as jnp
from jax import lax
from jax.experimental import pallas as pl
from jax.experimental.pallas import tpu as pltpu
from jax.experimental.pallas import tpu_sc as plsc

N = 10000
NP = 10240
D_IN = 128
H = 16
C = 2
E_RAW = 320000
E_TOT = E_RAW + N
NC = 2
NS = 16
L = 16
NW = NC * NS
K = 1536
CHUNKS = 7
T = K * CHUNKS
E_PAD = T * NW
RPT = NP // NS
BN = 256


def _tc_a_body(x_ref, w_ref, av_ref, dv_ref,
               h_ref, as_ref, ad_ref, m_ref):
    h = jnp.dot(x_ref[...], w_ref[...], preferred_element_type=jnp.float32)
    h_ref[...] = h
    a_s = jnp.sum(h * av_ref[...], axis=1, keepdims=True)
    a_d = jnp.sum(h * dv_ref[...], axis=1, keepdims=True)
    as_ref[...] = a_s
    ad_ref[...] = a_d
    m_ref[0, 0] = jnp.maximum(jnp.max(a_s) + jnp.max(a_d), 0.0)


def _tc_a(xp, W1, a_src1, a_dst1):
    return pl.pallas_call(
        _tc_a_body,
        out_specs=[
            pl.BlockSpec((NP, H), lambda: (0, 0)),
            pl.BlockSpec((NP, 1), lambda: (0, 0)),
            pl.BlockSpec((NP, 1), lambda: (0, 0)),
            pl.BlockSpec(memory_space=pltpu.SMEM),
        ],
        out_shape=[
            jax.ShapeDtypeStruct((NP, H), jnp.float32),
            jax.ShapeDtypeStruct((NP, 1), jnp.float32),
            jax.ShapeDtypeStruct((NP, 1), jnp.float32),
            jax.ShapeDtypeStruct((1, 1), jnp.float32),
        ],
    )(xp, W1, a_src1.reshape(1, H), a_dst1.reshape(1, H))


def _sc1(srcp, dstp, as1, ad1, h1, mvec):
    mesh = plsc.VectorSubcoreMesh(core_axis_name="c", subcore_axis_name="s",
                                  num_cores=NC, num_subcores=NS)

    @functools.partial(
        pl.kernel,
        out_type=[
            jax.ShapeDtypeStruct((NC, NP, H), jnp.float32),
            jax.ShapeDtypeStruct((NC, NP), jnp.float32),
        ],
        mesh=mesh,
        compiler_params=pltpu.CompilerParams(needs_layout_passes=False, use_tc_tiling_on_sc=False),
        scratch_types=[
            pltpu.VMEM((NP,), jnp.float32),
            pltpu.VMEM((NP,), jnp.float32),
            pltpu.VMEM((K,), jnp.int32),
            pltpu.VMEM((K,), jnp.int32),
            pltpu.VMEM((K,), jnp.float32),
            pltpu.VMEM((K, H), jnp.float32),
            pltpu.VMEM((L,), jnp.float32),
            pltpu.VMEM_SHARED((NP, H), jnp.float32),
            pltpu.VMEM_SHARED((NP,), jnp.float32),
            pltpu.SemaphoreType.DMA,
        ],
    )
    def k(src_hbm, dst_hbm, as_hbm, ad_hbm, h_hbm, m_hbm, num_out, s_out,
          as_v, ad_v, src_v, dst_v, p_v, rows_v, m_v, num_sh, s_sh, sem):
        cid = lax.axis_index("c")
        sid = lax.axis_index("s")
        wid = cid * NS + sid

        pltpu.sync_copy(as_hbm, as_v)
        pltpu.sync_copy(ad_hbm, ad_v)
        pltpu.sync_copy(m_hbm, m_v)
        mvec_r = m_v[...]

        def zrow(i, _):
            rows_v[i, :] = jnp.zeros((H,), jnp.float32)
            return 0
        lax.fori_loop(0, RPT, zrow, 0)

        def zp(i, _):
            p_v[pl.ds(i * L, L)] = jnp.zeros((L,), jnp.float32)
            return 0
        lax.fori_loop(0, RPT // L, zp, 0)

        pltpu.sync_copy(rows_v.at[pl.ds(0, RPT)],
                        num_sh.at[pl.ds(sid * RPT, RPT)])
        pltpu.sync_copy(p_v.at[pl.ds(0, RPT)],
                        s_sh.at[pl.ds(sid * RPT, RPT)])
        plsc.subcore_barrier()

        base = wid * T

        def chunk(ci, _):
            off = base + ci * K
            pltpu.sync_copy(src_hbm.at[pl.ds(off, K)], src_v)
            pltpu.sync_copy(dst_hbm.at[pl.ds(off, K)], dst_v)
            pltpu.async_copy(h_hbm.at[src_v], rows_v, sem).wait()

            def grp(g, _):
                sidx = src_v[pl.ds(g * L, L)]
                didx = dst_v[pl.ds(g * L, L)]
                e = plsc.load_gather(as_v, [sidx]) + plsc.load_gather(ad_v, [didx])
                e = jnp.maximum(e, 0.2 * e)
                p = jnp.exp(e - mvec_r)
                gid = off + g * L + lax.iota(jnp.int32, L)
                p = jnp.where(gid < E_TOT, p, 0.0)
                p_v[pl.ds(g * L, L)] = p
                for j in range(L):
                    rows_v[g * L + j, :] = rows_v[g * L + j, :] * p[j]
                return 0
            lax.fori_loop(0, K // L, grp, 0)

            pltpu.sync_copy(rows_v, num_sh.at[dst_v], add=True)
            pltpu.sync_copy(p_v, s_sh.at[dst_v], add=True)
            return 0
        lax.fori_loop(0, CHUNKS, chunk, 0)

        plsc.subcore_barrier()
        pltpu.sync_copy(num_sh.at[pl.ds(sid * RPT, RPT)],
                        num_out.at[cid, pl.ds(sid * RPT, RPT)])
        pltpu.sync_copy(s_sh.at[pl.ds(sid * RPT, RPT)],
                        s_out.at[cid, pl.ds(sid * RPT, RPT)])

    return k(srcp, dstp, as1, ad1, h1, mvec)


def _tc_b_body(num_ref, s_ref, b1_ref, w2_ref, a2_ref,
               h0_ref, h1_ref, as2_ref, ad2_ref, m_ref):
    num = num_ref[0] + num_ref[1]
    s = s_ref[0] + s_ref[1]
    s = jnp.where(s > 0.0, s, 1.0)
    out1 = num / s + b1_ref[...]
    h0 = jnp.sum(out1 * w2_ref[0:1, :], axis=1, keepdims=True)
    h1 = jnp.sum(out1 * w2_ref[1:2, :], axis=1, keepdims=True)
    h0_ref[...] = h0
    h1_ref[...] = h1
    a_s = h0 * a2_ref[0, 0] + h1 * a2_ref[0, 1]
    a_d = h0 * a2_ref[0, 2] + h1 * a2_ref[0, 3]
    as2_ref[...] = a_s
    ad2_ref[...] = a_d
    m_ref[0, 0] = jnp.maximum(jnp.max(a_s) + jnp.max(a_d), 0.0)


def _tc_b(num_p, s_p, b1, W2, a_src2, a_dst2):
    w2t = W2.T.reshape(2, H)
    a2 = jnp.concatenate([a_src2, a_dst2]).reshape(1, 4)
    return pl.pallas_call(
        _tc_b_body,
        in_specs=[
            pl.BlockSpec((NC, NP, H), lambda: (0, 0, 0)),
            pl.BlockSpec((NC, NP, 1), lambda: (0, 0, 0)),
            pl.BlockSpec((1, H), lambda: (0, 0)),
            pl.BlockSpec((2, H), lambda: (0, 0)),
            pl.BlockSpec(memory_space=pltpu.SMEM),
        ],
        out_specs=[
            pl.BlockSpec((NP, 1), lambda: (0, 0)),
            pl.BlockSpec((NP, 1), lambda: (0, 0)),
            pl.BlockSpec((NP, 1), lambda: (0, 0)),
            pl.BlockSpec((NP, 1), lambda: (0, 0)),
            pl.BlockSpec(memory_space=pltpu.SMEM),
        ],
        out_shape=[
            jax.ShapeDtypeStruct((NP, 1), jnp.float32),
            jax.ShapeDtypeStruct((NP, 1), jnp.float32),
            jax.ShapeDtypeStruct((NP, 1), jnp.float32),
            jax.ShapeDtypeStruct((NP, 1), jnp.float32),
            jax.ShapeDtypeStruct((1, 1), jnp.float32),
        ],
    )(num_p, s_p.reshape(NC, NP, 1), b1.reshape(1, H), w2t, a2)


def _sc2(srcp, dstp, as2, ad2, h2c0, h2c1, mvec):
    mesh = plsc.VectorSubcoreMesh(core_axis_name="c", subcore_axis_name="s",
                                  num_cores=NC, num_subcores=NS)

    @functools.partial(
        pl.kernel,
        out_type=[
            jax.ShapeDtypeStruct((NC, NP), jnp.float32),
            jax.ShapeDtypeStruct((NC, NP), jnp.float32),
            jax.ShapeDtypeStruct((NC, NP), jnp.float32),
        ],
        mesh=mesh,
        compiler_params=pltpu.CompilerParams(needs_layout_passes=False, use_tc_tiling_on_sc=False),
        scratch_types=[
            pltpu.VMEM((NP,), jnp.float32),
            pltpu.VMEM((NP,), jnp.float32),
            pltpu.VMEM((NP,), jnp.float32),
            pltpu.VMEM((NP,), jnp.float32),
            pltpu.VMEM((K,), jnp.int32),
            pltpu.VMEM((K,), jnp.int32),
            pltpu.VMEM((K,), jnp.float32),
            pltpu.VMEM((K,), jnp.float32),
            pltpu.VMEM((K,), jnp.float32),
            pltpu.VMEM((L,), jnp.float32),
            pltpu.VMEM_SHARED((NP,), jnp.float32),
            pltpu.VMEM_SHARED((NP,), jnp.float32),
            pltpu.VMEM_SHARED((NP,), jnp.float32),
        ],
    )
    def k(src_hbm, dst_hbm, as_hbm, ad_hbm, c0_hbm, c1_hbm, m_hbm,
          n0_out, n1_out, s_out,
          as_v, ad_v, c0_v, c1_v, src_v, dst_v, p_v, v0_v, v1_v, m_v,
          n0_sh, n1_sh, s_sh):
        cid = lax.axis_index("c")
        sid = lax.axis_index("s")
        wid = cid * NS + sid

        pltpu.sync_copy(as_hbm, as_v)
        pltpu.sync_copy(ad_hbm, ad_v)
        pltpu.sync_copy(c0_hbm, c0_v)
        pltpu.sync_copy(c1_hbm, c1_v)
        pltpu.sync_copy(m_hbm, m_v)
        mvec_r = m_v[...]

        def zp(i, _):
            p_v[pl.ds(i * L, L)] = jnp.zeros((L,), jnp.float32)
            return 0
        lax.fori_loop(0, RPT // L, zp, 0)
        pltpu.sync_copy(p_v.at[pl.ds(0, RPT)], n0_sh.at[pl.ds(sid * RPT, RPT)])
        pltpu.sync_copy(p_v.at[pl.ds(0, RPT)], n1_sh.at[pl.ds(sid * RPT, RPT)])
        pltpu.sync_copy(p_v.at[pl.ds(0, RPT)], s_sh.at[pl.ds(sid * RPT, RPT)])
        plsc.subcore_barrier()

        base = wid * T

        def chunk(ci, _):
            off = base + ci * K
            pltpu.sync_copy(src_hbm.at[pl.ds(off, K)], src_v)
            pltpu.sync_copy(dst_hbm.at[pl.ds(off, K)], dst_v)

            def grp(g, _):
                sidx = src_v[pl.ds(g * L, L)]
                didx = dst_v[pl.ds(g * L, L)]
                e = plsc.load_gather(as_v, [sidx]) + plsc.load_gather(ad_v, [didx])
                e = jnp.maximum(e, 0.2 * e)
                p = jnp.exp(e - mvec_r)
                gid = off + g * L + lax.iota(jnp.int32, L)
                p = jnp.where(gid < E_TOT, p, 0.0)
                p_v[pl.ds(g * L, L)] = p
                v0_v[pl.ds(g * L, L)] = p * plsc.load_gather(c0_v, [sidx])
                v1_v[pl.ds(g * L, L)] = p * plsc.load_gather(c1_v, [sidx])
                return 0
            lax.fori_loop(0, K // L, grp, 0)

            pltpu.sync_copy(v0_v, n0_sh.at[dst_v], add=True)
            pltpu.sync_copy(v1_v, n1_sh.at[dst_v], add=True)
            pltpu.sync_copy(p_v, s_sh.at[dst_v], add=True)
            return 0
        lax.fori_loop(0, CHUNKS, chunk, 0)

        plsc.subcore_barrier()
        pltpu.sync_copy(n0_sh.at[pl.ds(sid * RPT, RPT)],
                        n0_out.at[cid, pl.ds(sid * RPT, RPT)])
        pltpu.sync_copy(n1_sh.at[pl.ds(sid * RPT, RPT)],
                        n1_out.at[cid, pl.ds(sid * RPT, RPT)])
        pltpu.sync_copy(s_sh.at[pl.ds(sid * RPT, RPT)],
                        s_out.at[cid, pl.ds(sid * RPT, RPT)])

    return k(srcp, dstp, as2, ad2, h2c0, h2c1, mvec)


def _tc_c_body(n0_ref, n1_ref, s_ref, b2_ref, out_ref, mn_ref, mx_ref):
    n0 = n0_ref[0] + n0_ref[1]
    n1 = n1_ref[0] + n1_ref[1]
    s = s_ref[0] + s_ref[1]
    s = jnp.where(s > 0.0, s, 1.0)
    o = jnp.concatenate([n0 / s + b2_ref[0, 0], n1 / s + b2_ref[0, 1]], axis=1)
    out_ref[...] = o
    rid = lax.broadcasted_iota(jnp.int32, (NP, C), 0)
    valid = rid < N
    mn_ref[0, 0] = jnp.min(jnp.where(valid, o, jnp.inf))
    mx_ref[0, 0] = jnp.max(jnp.where(valid, o, -jnp.inf))


def _tc_c(n0_p, n1_p, s2_p, b2):
    return pl.pallas_call(
        _tc_c_body,
        in_specs=[
            pl.BlockSpec((NC, NP, 1), lambda: (0, 0, 0)),
            pl.BlockSpec((NC, NP, 1), lambda: (0, 0, 0)),
            pl.BlockSpec((NC, NP, 1), lambda: (0, 0, 0)),
            pl.BlockSpec(memory_space=pltpu.SMEM),
        ],
        out_specs=[
            pl.BlockSpec((NP, C), lambda: (0, 0)),
            pl.BlockSpec(memory_space=pltpu.SMEM),
            pl.BlockSpec(memory_space=pltpu.SMEM),
        ],
        out_shape=[
            jax.ShapeDtypeStruct((NP, C), jnp.float32),
            jax.ShapeDtypeStruct((1, 1), jnp.float32),
            jax.ShapeDtypeStruct((1, 1), jnp.float32),
        ],
    )(n0_p.reshape(NC, NP, 1), n1_p.reshape(NC, NP, 1),
      s2_p.reshape(NC, NP, 1), b2.reshape(1, 2))


def _tc_d_body(o_ref, mn_ref, mx_ref, y_ref):
    mn = mn_ref[0, 0]
    mx = mx_ref[0, 0]
    y_ref[...] = 2.0 * ((o_ref[...] - mn) / (mx - mn)) - 1.0


def _tc_d(out2, mn, mx):
    return pl.pallas_call(
        _tc_d_body,
        in_specs=[
            pl.BlockSpec((NP, C), lambda: (0, 0)),
            pl.BlockSpec(memory_space=pltpu.SMEM),
            pl.BlockSpec(memory_space=pltpu.SMEM),
        ],
        out_shape=jax.ShapeDtypeStruct((NP, C), jnp.float32),
    )(out2, mn, mx)


def kernel(x, edge_index, W1, a_src1, a_dst1, b1, W2, a_src2, a_dst2, b2):
    xp = jnp.zeros((NP, D_IN), jnp.float32).at[:N].set(x)
    loop = jnp.arange(N, dtype=jnp.int32)
    pad = jnp.arange(E_PAD - E_TOT, dtype=jnp.int32) % N
    srcp = jnp.concatenate([edge_index[0].astype(jnp.int32), loop, pad])
    dstp = jnp.concatenate([edge_index[1].astype(jnp.int32), loop, pad])

    h1, as1, ad1, m1 = _tc_a(xp, W1, a_src1, a_dst1)
    mvec1 = jnp.broadcast_to(m1.reshape(1), (L,))
    num_p, s_p = _sc1(srcp, dstp, as1.reshape(NP), ad1.reshape(NP), h1, mvec1)

    h2c0, h2c1, as2, ad2, m2 = _tc_b(num_p, s_p, b1, W2, a_src2, a_dst2)
    mvec2 = jnp.broadcast_to(m2.reshape(1), (L,))
    n0_p, n1_p, s2_p = _sc2(srcp, dstp, as2.reshape(NP), ad2.reshape(NP),
                            h2c0.reshape(NP), h2c1.reshape(NP), mvec2)

    out2, mn, mx = _tc_c(n0_p, n1_p, s2_p, b2)
    y = _tc_d(out2, mn, mx)
    return y[:N]


# speedup vs baseline: 88.6899x; 1.2278x over previous
import functools

import jax
import jax.numpy as jnp
from jax import lax
from jax.experimental import pallas as pl
from jax.experimental.pallas import tpu as pltpu
from jax.experimental.pallas import tpu_sc as plsc

N = 10000
NP = 10240
D_IN = 128
H = 16
C = 2
E_RAW = 320000
E_TOT = E_RAW + N
NC = 2
NS = 16
L = 16
NW = NC * NS
K = 1536
CHUNKS = 7
T = K * CHUNKS
E_PAD = T * NW
RPT = NP // NS
HA = 32
BN = 256


def _tc_a_body(x_ref, w_ref, av_ref, dv_ref,
               h_ref, as_ref, ad_ref, m_ref):
    h = jnp.dot(x_ref[...], w_ref[...], preferred_element_type=jnp.float32)
    h_ref[...] = h
    a_s = jnp.sum(h * av_ref[...], axis=1, keepdims=True)
    a_d = jnp.sum(h * dv_ref[...], axis=1, keepdims=True)
    as_ref[...] = a_s
    ad_ref[...] = a_d
    m_ref[0, 0] = jnp.maximum(jnp.max(a_s) + jnp.max(a_d), 0.0)


def _tc_a(xp, W1, a_src1, a_dst1):
    return pl.pallas_call(
        _tc_a_body,
        out_specs=[
            pl.BlockSpec((NP, H), lambda: (0, 0)),
            pl.BlockSpec((NP, 1), lambda: (0, 0)),
            pl.BlockSpec((NP, 1), lambda: (0, 0)),
            pl.BlockSpec(memory_space=pltpu.SMEM),
        ],
        out_shape=[
            jax.ShapeDtypeStruct((NP, H), jnp.float32),
            jax.ShapeDtypeStruct((NP, 1), jnp.float32),
            jax.ShapeDtypeStruct((NP, 1), jnp.float32),
            jax.ShapeDtypeStruct((1, 1), jnp.float32),
        ],
    )(xp, W1, a_src1.reshape(1, H), a_dst1.reshape(1, H))


def _sc1(srcp, dstp, as1, ad1, h1, mvec):
    mesh = plsc.VectorSubcoreMesh(core_axis_name="c", subcore_axis_name="s",
                                  num_cores=NC, num_subcores=NS)

    @functools.partial(
        pl.kernel,
        out_type=[
            jax.ShapeDtypeStruct((NC, NP, H), jnp.float32),
            jax.ShapeDtypeStruct((NC, NP), jnp.float32),
        ],
        mesh=mesh,
        compiler_params=pltpu.CompilerParams(needs_layout_passes=False, use_tc_tiling_on_sc=False),
        scratch_types=[
            pltpu.VMEM((NP,), jnp.float32),
            pltpu.VMEM((NP,), jnp.float32),
            pltpu.VMEM((K,), jnp.int32),
            pltpu.VMEM((K,), jnp.int32),
            pltpu.VMEM((K,), jnp.float32),
            pltpu.VMEM((K, H), jnp.float32),
            pltpu.VMEM((L,), jnp.float32),
            pltpu.VMEM_SHARED((NP, H), jnp.float32),
            pltpu.VMEM_SHARED((NP,), jnp.float32),
            pltpu.SemaphoreType.DMA,
        ],
    )
    def k(src_hbm, dst_hbm, as_hbm, ad_hbm, h_hbm, m_hbm, num_out, s_out,
          as_v, ad_v, src_v, dst_v, p_v, rows_v, m_v, num_sh, s_sh, sem):
        cid = lax.axis_index("c")
        sid = lax.axis_index("s")
        wid = cid * NS + sid

        pltpu.sync_copy(as_hbm, as_v)
        pltpu.sync_copy(ad_hbm, ad_v)
        pltpu.sync_copy(m_hbm, m_v)
        mvec_r = m_v[...]

        def zrow(i, _):
            rows_v[i, :] = jnp.zeros((H,), jnp.float32)
            return 0
        lax.fori_loop(0, RPT, zrow, 0)

        def zp(i, _):
            p_v[pl.ds(i * L, L)] = jnp.zeros((L,), jnp.float32)
            return 0
        lax.fori_loop(0, RPT // L, zp, 0)

        pltpu.sync_copy(rows_v.at[pl.ds(0, RPT)],
                        num_sh.at[pl.ds(sid * RPT, RPT)])
        pltpu.sync_copy(p_v.at[pl.ds(0, RPT)],
                        s_sh.at[pl.ds(sid * RPT, RPT)])
        plsc.subcore_barrier()

        base = wid * T

        def chunk(ci, _):
            off = base + ci * K
            pltpu.sync_copy(src_hbm.at[pl.ds(off, K)], src_v)
            pltpu.sync_copy(dst_hbm.at[pl.ds(off, K)], dst_v)
            pltpu.async_copy(h_hbm.at[src_v], rows_v, sem).wait()

            def grp(g, _):
                sidx = src_v[pl.ds(g * L, L)]
                didx = dst_v[pl.ds(g * L, L)]
                e = plsc.load_gather(as_v, [sidx]) + plsc.load_gather(ad_v, [didx])
                e = jnp.maximum(e, 0.2 * e)
                p = jnp.exp(e - mvec_r)
                gid = off + g * L + lax.iota(jnp.int32, L)
                p = jnp.where(gid < E_TOT, p, 0.0)
                p_v[pl.ds(g * L, L)] = p
                for j in range(L):
                    rows_v[g * L + j, :] = rows_v[g * L + j, :] * p[j]
                return 0
            lax.fori_loop(0, K // L, grp, 0)

            pltpu.sync_copy(rows_v, num_sh.at[dst_v], add=True)
            pltpu.sync_copy(p_v, s_sh.at[dst_v], add=True)
            return 0
        lax.fori_loop(0, CHUNKS, chunk, 0)

        plsc.subcore_barrier()
        pltpu.sync_copy(num_sh.at[pl.ds(sid * RPT, RPT)],
                        num_out.at[cid, pl.ds(sid * RPT, RPT)])
        pltpu.sync_copy(s_sh.at[pl.ds(sid * RPT, RPT)],
                        s_out.at[cid, pl.ds(sid * RPT, RPT)])

    return k(srcp, dstp, as1, ad1, h1, mvec)


def _tc_b_body(num_ref, s_ref, b1_ref, w2_ref, a2_ref,
               h0_ref, h1_ref, as2_ref, ad2_ref, m_ref):
    num = num_ref[0] + num_ref[1]
    s = s_ref[0] + s_ref[1]
    s = jnp.where(s > 0.0, s, 1.0)
    out1 = num / s + b1_ref[...]
    h0 = jnp.sum(out1 * w2_ref[0:1, :], axis=1, keepdims=True)
    h1 = jnp.sum(out1 * w2_ref[1:2, :], axis=1, keepdims=True)
    h0_ref[...] = h0
    h1_ref[...] = h1
    a_s = h0 * a2_ref[0, 0] + h1 * a2_ref[0, 1]
    a_d = h0 * a2_ref[0, 2] + h1 * a2_ref[0, 3]
    as2_ref[...] = a_s
    ad2_ref[...] = a_d
    m_ref[0, 0] = jnp.maximum(jnp.max(a_s) + jnp.max(a_d), 0.0)


def _tc_b(num_p, s_p, b1, W2, a_src2, a_dst2):
    w2t = W2.T.reshape(2, H)
    a2 = jnp.concatenate([a_src2, a_dst2]).reshape(1, 4)
    return pl.pallas_call(
        _tc_b_body,
        in_specs=[
            pl.BlockSpec((NC, NP, H), lambda: (0, 0, 0)),
            pl.BlockSpec((NC, NP, 1), lambda: (0, 0, 0)),
            pl.BlockSpec((1, H), lambda: (0, 0)),
            pl.BlockSpec((2, H), lambda: (0, 0)),
            pl.BlockSpec(memory_space=pltpu.SMEM),
        ],
        out_specs=[
            pl.BlockSpec((NP, 1), lambda: (0, 0)),
            pl.BlockSpec((NP, 1), lambda: (0, 0)),
            pl.BlockSpec((NP, 1), lambda: (0, 0)),
            pl.BlockSpec((NP, 1), lambda: (0, 0)),
            pl.BlockSpec(memory_space=pltpu.SMEM),
        ],
        out_shape=[
            jax.ShapeDtypeStruct((NP, 1), jnp.float32),
            jax.ShapeDtypeStruct((NP, 1), jnp.float32),
            jax.ShapeDtypeStruct((NP, 1), jnp.float32),
            jax.ShapeDtypeStruct((NP, 1), jnp.float32),
            jax.ShapeDtypeStruct((1, 1), jnp.float32),
        ],
    )(num_p, s_p.reshape(NC, NP, 1), b1.reshape(1, H), w2t, a2)


def _sc2(srcp, dstp, as2, ad2, h2c0, h2c1, mvec):
    mesh = plsc.VectorSubcoreMesh(core_axis_name="c", subcore_axis_name="s",
                                  num_cores=NC, num_subcores=NS)

    @functools.partial(
        pl.kernel,
        out_type=[
            jax.ShapeDtypeStruct((NC, NP), jnp.float32),
            jax.ShapeDtypeStruct((NC, NP), jnp.float32),
            jax.ShapeDtypeStruct((NC, NP), jnp.float32),
        ],
        mesh=mesh,
        compiler_params=pltpu.CompilerParams(needs_layout_passes=False, use_tc_tiling_on_sc=False),
        scratch_types=[
            pltpu.VMEM((NP,), jnp.float32),
            pltpu.VMEM((NP,), jnp.float32),
            pltpu.VMEM((NP,), jnp.float32),
            pltpu.VMEM((NP,), jnp.float32),
            pltpu.VMEM((K,), jnp.int32),
            pltpu.VMEM((K,), jnp.int32),
            pltpu.VMEM((K,), jnp.float32),
            pltpu.VMEM((K,), jnp.float32),
            pltpu.VMEM((K,), jnp.float32),
            pltpu.VMEM((L,), jnp.float32),
            pltpu.VMEM_SHARED((NP,), jnp.float32),
            pltpu.VMEM_SHARED((NP,), jnp.float32),
            pltpu.VMEM_SHARED((NP,), jnp.float32),
        ],
    )
    def k(src_hbm, dst_hbm, as_hbm, ad_hbm, c0_hbm, c1_hbm, m_hbm,
          n0_out, n1_out, s_out,
          as_v, ad_v, c0_v, c1_v, src_v, dst_v, p_v, v0_v, v1_v, m_v,
          n0_sh, n1_sh, s_sh):
        cid = lax.axis_index("c")
        sid = lax.axis_index("s")
        wid = cid * NS + sid

        pltpu.sync_copy(as_hbm, as_v)
        pltpu.sync_copy(ad_hbm, ad_v)
        pltpu.sync_copy(c0_hbm, c0_v)
        pltpu.sync_copy(c1_hbm, c1_v)
        pltpu.sync_copy(m_hbm, m_v)
        mvec_r = m_v[...]

        def zp(i, _):
            p_v[pl.ds(i * L, L)] = jnp.zeros((L,), jnp.float32)
            return 0
        lax.fori_loop(0, RPT // L, zp, 0)
        pltpu.sync_copy(p_v.at[pl.ds(0, RPT)], n0_sh.at[pl.ds(sid * RPT, RPT)])
        pltpu.sync_copy(p_v.at[pl.ds(0, RPT)], n1_sh.at[pl.ds(sid * RPT, RPT)])
        pltpu.sync_copy(p_v.at[pl.ds(0, RPT)], s_sh.at[pl.ds(sid * RPT, RPT)])
        plsc.subcore_barrier()

        base = wid * T

        def chunk(ci, _):
            off = base + ci * K
            pltpu.sync_copy(src_hbm.at[pl.ds(off, K)], src_v)
            pltpu.sync_copy(dst_hbm.at[pl.ds(off, K)], dst_v)

            def grp(g, _):
                sidx = src_v[pl.ds(g * L, L)]
                didx = dst_v[pl.ds(g * L, L)]
                e = plsc.load_gather(as_v, [sidx]) + plsc.load_gather(ad_v, [didx])
                e = jnp.maximum(e, 0.2 * e)
                p = jnp.exp(e - mvec_r)
                gid = off + g * L + lax.iota(jnp.int32, L)
                p = jnp.where(gid < E_TOT, p, 0.0)
                p_v[pl.ds(g * L, L)] = p
                v0_v[pl.ds(g * L, L)] = p * plsc.load_gather(c0_v, [sidx])
                v1_v[pl.ds(g * L, L)] = p * plsc.load_gather(c1_v, [sidx])
                return 0
            lax.fori_loop(0, K // L, grp, 0)

            pltpu.sync_copy(v0_v, n0_sh.at[dst_v], add=True)
            pltpu.sync_copy(v1_v, n1_sh.at[dst_v], add=True)
            pltpu.sync_copy(p_v, s_sh.at[dst_v], add=True)
            return 0
        lax.fori_loop(0, CHUNKS, chunk, 0)

        plsc.subcore_barrier()
        pltpu.sync_copy(n0_sh.at[pl.ds(sid * RPT, RPT)],
                        n0_out.at[cid, pl.ds(sid * RPT, RPT)])
        pltpu.sync_copy(n1_sh.at[pl.ds(sid * RPT, RPT)],
                        n1_out.at[cid, pl.ds(sid * RPT, RPT)])
        pltpu.sync_copy(s_sh.at[pl.ds(sid * RPT, RPT)],
                        s_out.at[cid, pl.ds(sid * RPT, RPT)])

    return k(srcp, dstp, as2, ad2, h2c0, h2c1, mvec)


NR = NP // 128


def _tc_c_body(n0_ref, n1_ref, s_ref, b2_ref, y_ref):
    n0 = n0_ref[0] + n0_ref[1]
    n1 = n1_ref[0] + n1_ref[1]
    s = s_ref[0] + s_ref[1]
    s = jnp.where(s > 0.0, s, 1.0)
    o0 = n0 / s + b2_ref[0, 0]
    o1 = n1 / s + b2_ref[0, 1]
    nid = (lax.broadcasted_iota(jnp.int32, (NR, 128), 0) * 128
           + lax.broadcasted_iota(jnp.int32, (NR, 128), 1))
    valid = nid < N
    mn = jnp.minimum(jnp.min(jnp.where(valid, o0, jnp.inf)),
                     jnp.min(jnp.where(valid, o1, jnp.inf)))
    mx = jnp.maximum(jnp.max(jnp.where(valid, o0, -jnp.inf)),
                     jnp.max(jnp.where(valid, o1, -jnp.inf)))
    rng = mx - mn
    y_ref[0] = 2.0 * ((o0 - mn) / rng) - 1.0
    y_ref[1] = 2.0 * ((o1 - mn) / rng) - 1.0


def _tc_c(n0_p, n1_p, s2_p, b2):
    return pl.pallas_call(
        _tc_c_body,
        in_specs=[
            pl.BlockSpec((NC, NR, 128), lambda: (0, 0, 0)),
            pl.BlockSpec((NC, NR, 128), lambda: (0, 0, 0)),
            pl.BlockSpec((NC, NR, 128), lambda: (0, 0, 0)),
            pl.BlockSpec(memory_space=pltpu.SMEM),
        ],
        out_specs=pl.BlockSpec((C, NR, 128), lambda: (0, 0, 0)),
        out_shape=jax.ShapeDtypeStruct((C, NR, 128), jnp.float32),
    )(n0_p.reshape(NC, NR, 128), n1_p.reshape(NC, NR, 128),
      s2_p.reshape(NC, NR, 128), b2.reshape(1, 2))


def kernel(x, edge_index, W1, a_src1, a_dst1, b1, W2, a_src2, a_dst2, b2):
    xp = jnp.zeros((NP, D_IN), jnp.float32).at[:N].set(x)
    loop = jnp.arange(N, dtype=jnp.int32)
    pad = jnp.arange(E_PAD - E_TOT, dtype=jnp.int32) % N
    srcp = jnp.concatenate([edge_index[0].astype(jnp.int32), loop, pad])
    dstp = jnp.concatenate([edge_index[1].astype(jnp.int32), loop, pad])

    h1, as1, ad1, m1 = _tc_a(xp, W1, a_src1, a_dst1)
    mvec1 = jnp.broadcast_to(m1.reshape(1), (L,))
    num_p, s_p = _sc1(srcp, dstp, as1.reshape(NP), ad1.reshape(NP), h1, mvec1)

    h2c0, h2c1, as2, ad2, m2 = _tc_b(num_p, s_p, b1, W2, a_src2, a_dst2)
    mvec2 = jnp.broadcast_to(m2.reshape(1), (L,))
    n0_p, n1_p, s2_p = _sc2(srcp, dstp, as2.reshape(NP), ad2.reshape(NP),
                            h2c0.reshape(NP), h2c1.reshape(NP), mvec2)

    y = _tc_c(n0_p, n1_p, s2_p, b2)
    return jnp.stack([y.reshape(C, NP)[0, :N], y.reshape(C, NP)[1, :N]], axis=1)


# speedup vs baseline: 101.5623x; 1.1451x over previous
import functools

import jax
import jax.numpy as jnp
from jax import lax
from jax.experimental import pallas as pl
from jax.experimental.pallas import tpu as pltpu
from jax.experimental.pallas import tpu_sc as plsc

N = 10000
NP = 10240
D_IN = 128
H = 16
C = 2
E_RAW = 320000
E_TOT = E_RAW + N
NC = 2
NS = 16
L = 16
NW = NC * NS
K = 1536
CHUNKS = 7
T = K * CHUNKS
E_PAD = T * NW
RPT = NP // NS
HA = 32
BN = 256


def _tc_a_body(x_ref, w_ref, av_ref, dv_ref,
               h_ref, as_ref, ad_ref, m_ref):
    h = jnp.dot(x_ref[...], w_ref[...], preferred_element_type=jnp.float32)
    h_ref[...] = h
    a_s = jnp.sum(h * av_ref[...], axis=1, keepdims=True)
    a_d = jnp.sum(h * dv_ref[...], axis=1, keepdims=True)
    as_ref[...] = a_s
    ad_ref[...] = a_d
    m_ref[0, 0] = jnp.maximum(jnp.max(a_s) + jnp.max(a_d), 0.0)


def _tc_a(xp, W1, a_src1, a_dst1):
    return pl.pallas_call(
        _tc_a_body,
        out_specs=[
            pl.BlockSpec((NP, H), lambda: (0, 0)),
            pl.BlockSpec((NP, 1), lambda: (0, 0)),
            pl.BlockSpec((NP, 1), lambda: (0, 0)),
            pl.BlockSpec(memory_space=pltpu.SMEM),
        ],
        out_shape=[
            jax.ShapeDtypeStruct((NP, H), jnp.float32),
            jax.ShapeDtypeStruct((NP, 1), jnp.float32),
            jax.ShapeDtypeStruct((NP, 1), jnp.float32),
            jax.ShapeDtypeStruct((1, 1), jnp.float32),
        ],
    )(xp, W1, a_src1.reshape(1, H), a_dst1.reshape(1, H))


def _sc1(srcp, dstp, as1, ad1, h1, mvec):
    mesh = plsc.VectorSubcoreMesh(core_axis_name="c", subcore_axis_name="s",
                                  num_cores=NC, num_subcores=NS)

    @functools.partial(
        pl.kernel,
        out_type=[
            jax.ShapeDtypeStruct((NC, NP, H), jnp.float32),
            jax.ShapeDtypeStruct((NC, NP), jnp.float32),
        ],
        mesh=mesh,
        compiler_params=pltpu.CompilerParams(needs_layout_passes=False, use_tc_tiling_on_sc=False),
        scratch_types=[
            pltpu.VMEM((NP,), jnp.float32),
            pltpu.VMEM((NP,), jnp.float32),
            pltpu.VMEM((K,), jnp.int32),
            pltpu.VMEM((K,), jnp.int32),
            pltpu.VMEM((K,), jnp.float32),
            pltpu.VMEM((K, H), jnp.float32),
            pltpu.VMEM((L,), jnp.float32),
            pltpu.VMEM_SHARED((NP, H), jnp.float32),
            pltpu.VMEM_SHARED((NP,), jnp.float32),
            pltpu.SemaphoreType.DMA,
        ],
    )
    def k(src_hbm, dst_hbm, as_hbm, ad_hbm, h_hbm, m_hbm, num_out, s_out,
          as_v, ad_v, src_v, dst_v, p_v, rows_v, m_v, num_sh, s_sh, sem):
        cid = lax.axis_index("c")
        sid = lax.axis_index("s")
        wid = cid * NS + sid

        pltpu.sync_copy(as_hbm, as_v)
        pltpu.sync_copy(ad_hbm, ad_v)
        pltpu.sync_copy(m_hbm, m_v)
        mvec_r = m_v[...]

        def zrow(i, _):
            rows_v[i, :] = jnp.zeros((H,), jnp.float32)
            return 0
        lax.fori_loop(0, RPT, zrow, 0)

        def zp(i, _):
            p_v[pl.ds(i * L, L)] = jnp.zeros((L,), jnp.float32)
            return 0
        lax.fori_loop(0, RPT // L, zp, 0)

        pltpu.sync_copy(rows_v.at[pl.ds(0, RPT)],
                        num_sh.at[pl.ds(sid * RPT, RPT)])
        pltpu.sync_copy(p_v.at[pl.ds(0, RPT)],
                        s_sh.at[pl.ds(sid * RPT, RPT)])
        plsc.subcore_barrier()

        base = wid * T

        def chunk(ci, _):
            off = base + ci * K
            pltpu.sync_copy(src_hbm.at[pl.ds(off, K)], src_v)
            pltpu.sync_copy(dst_hbm.at[pl.ds(off, K)], dst_v)
            gdesc = pltpu.async_copy(h_hbm.at[src_v], rows_v, sem)

            @plsc.parallel_loop(0, K // L, unroll=2)
            def _(g):
                sidx = src_v[pl.ds(g * L, L)]
                didx = dst_v[pl.ds(g * L, L)]
                e = plsc.load_gather(as_v, [sidx]) + plsc.load_gather(ad_v, [didx])
                e = jnp.maximum(e, 0.2 * e)
                p = jnp.exp(e - mvec_r)
                gid = off + g * L + lax.iota(jnp.int32, L)
                p_v[pl.ds(g * L, L)] = jnp.where(gid < E_TOT, p, 0.0)

            gdesc.wait()

            @plsc.parallel_loop(0, K // L, unroll=2)
            def _(g):
                p = p_v[pl.ds(g * L, L)]
                for j in range(L):
                    rows_v[g * L + j, :] = rows_v[g * L + j, :] * p[j]

            pltpu.sync_copy(rows_v, num_sh.at[dst_v], add=True)
            pltpu.sync_copy(p_v, s_sh.at[dst_v], add=True)
            return 0
        lax.fori_loop(0, CHUNKS, chunk, 0)

        plsc.subcore_barrier()
        pltpu.sync_copy(num_sh.at[pl.ds(sid * RPT, RPT)],
                        num_out.at[cid, pl.ds(sid * RPT, RPT)])
        pltpu.sync_copy(s_sh.at[pl.ds(sid * RPT, RPT)],
                        s_out.at[cid, pl.ds(sid * RPT, RPT)])

    return k(srcp, dstp, as1, ad1, h1, mvec)


def _tc_b_body(num_ref, s_ref, b1_ref, w2_ref, a2_ref,
               h0_ref, h1_ref, as2_ref, ad2_ref, m_ref):
    num = num_ref[0] + num_ref[1]
    s = s_ref[0] + s_ref[1]
    s = jnp.where(s > 0.0, s, 1.0)
    out1 = num / s + b1_ref[...]
    h0 = jnp.sum(out1 * w2_ref[0:1, :], axis=1, keepdims=True)
    h1 = jnp.sum(out1 * w2_ref[1:2, :], axis=1, keepdims=True)
    h0_ref[...] = h0
    h1_ref[...] = h1
    a_s = h0 * a2_ref[0, 0] + h1 * a2_ref[0, 1]
    a_d = h0 * a2_ref[0, 2] + h1 * a2_ref[0, 3]
    as2_ref[...] = a_s
    ad2_ref[...] = a_d
    m_ref[0, 0] = jnp.maximum(jnp.max(a_s) + jnp.max(a_d), 0.0)


def _tc_b(num_p, s_p, b1, W2, a_src2, a_dst2):
    w2t = W2.T.reshape(2, H)
    a2 = jnp.concatenate([a_src2, a_dst2]).reshape(1, 4)
    return pl.pallas_call(
        _tc_b_body,
        in_specs=[
            pl.BlockSpec((NC, NP, H), lambda: (0, 0, 0)),
            pl.BlockSpec((NC, NP, 1), lambda: (0, 0, 0)),
            pl.BlockSpec((1, H), lambda: (0, 0)),
            pl.BlockSpec((2, H), lambda: (0, 0)),
            pl.BlockSpec(memory_space=pltpu.SMEM),
        ],
        out_specs=[
            pl.BlockSpec((NP, 1), lambda: (0, 0)),
            pl.BlockSpec((NP, 1), lambda: (0, 0)),
            pl.BlockSpec((NP, 1), lambda: (0, 0)),
            pl.BlockSpec((NP, 1), lambda: (0, 0)),
            pl.BlockSpec(memory_space=pltpu.SMEM),
        ],
        out_shape=[
            jax.ShapeDtypeStruct((NP, 1), jnp.float32),
            jax.ShapeDtypeStruct((NP, 1), jnp.float32),
            jax.ShapeDtypeStruct((NP, 1), jnp.float32),
            jax.ShapeDtypeStruct((NP, 1), jnp.float32),
            jax.ShapeDtypeStruct((1, 1), jnp.float32),
        ],
    )(num_p, s_p.reshape(NC, NP, 1), b1.reshape(1, H), w2t, a2)


def _sc2(srcp, dstp, as2, ad2, h2c0, h2c1, mvec):
    mesh = plsc.VectorSubcoreMesh(core_axis_name="c", subcore_axis_name="s",
                                  num_cores=NC, num_subcores=NS)

    @functools.partial(
        pl.kernel,
        out_type=[
            jax.ShapeDtypeStruct((NC, NP), jnp.float32),
            jax.ShapeDtypeStruct((NC, NP), jnp.float32),
            jax.ShapeDtypeStruct((NC, NP), jnp.float32),
        ],
        mesh=mesh,
        compiler_params=pltpu.CompilerParams(needs_layout_passes=False, use_tc_tiling_on_sc=False),
        scratch_types=[
            pltpu.VMEM((NP,), jnp.float32),
            pltpu.VMEM((NP,), jnp.float32),
            pltpu.VMEM((NP,), jnp.float32),
            pltpu.VMEM((NP,), jnp.float32),
            pltpu.VMEM((K,), jnp.int32),
            pltpu.VMEM((K,), jnp.int32),
            pltpu.VMEM((K,), jnp.float32),
            pltpu.VMEM((K,), jnp.float32),
            pltpu.VMEM((K,), jnp.float32),
            pltpu.VMEM((L,), jnp.float32),
            pltpu.VMEM_SHARED((NP,), jnp.float32),
            pltpu.VMEM_SHARED((NP,), jnp.float32),
            pltpu.VMEM_SHARED((NP,), jnp.float32),
        ],
    )
    def k(src_hbm, dst_hbm, as_hbm, ad_hbm, c0_hbm, c1_hbm, m_hbm,
          n0_out, n1_out, s_out,
          as_v, ad_v, c0_v, c1_v, src_v, dst_v, p_v, v0_v, v1_v, m_v,
          n0_sh, n1_sh, s_sh):
        cid = lax.axis_index("c")
        sid = lax.axis_index("s")
        wid = cid * NS + sid

        pltpu.sync_copy(as_hbm, as_v)
        pltpu.sync_copy(ad_hbm, ad_v)
        pltpu.sync_copy(c0_hbm, c0_v)
        pltpu.sync_copy(c1_hbm, c1_v)
        pltpu.sync_copy(m_hbm, m_v)
        mvec_r = m_v[...]

        def zp(i, _):
            p_v[pl.ds(i * L, L)] = jnp.zeros((L,), jnp.float32)
            return 0
        lax.fori_loop(0, RPT // L, zp, 0)
        pltpu.sync_copy(p_v.at[pl.ds(0, RPT)], n0_sh.at[pl.ds(sid * RPT, RPT)])
        pltpu.sync_copy(p_v.at[pl.ds(0, RPT)], n1_sh.at[pl.ds(sid * RPT, RPT)])
        pltpu.sync_copy(p_v.at[pl.ds(0, RPT)], s_sh.at[pl.ds(sid * RPT, RPT)])
        plsc.subcore_barrier()

        base = wid * T

        def chunk(ci, _):
            off = base + ci * K
            pltpu.sync_copy(src_hbm.at[pl.ds(off, K)], src_v)
            pltpu.sync_copy(dst_hbm.at[pl.ds(off, K)], dst_v)

            @plsc.parallel_loop(0, K // L, unroll=2)
            def _(g):
                sidx = src_v[pl.ds(g * L, L)]
                didx = dst_v[pl.ds(g * L, L)]
                e = plsc.load_gather(as_v, [sidx]) + plsc.load_gather(ad_v, [didx])
                e = jnp.maximum(e, 0.2 * e)
                p = jnp.exp(e - mvec_r)
                gid = off + g * L + lax.iota(jnp.int32, L)
                p = jnp.where(gid < E_TOT, p, 0.0)
                p_v[pl.ds(g * L, L)] = p
                v0_v[pl.ds(g * L, L)] = p * plsc.load_gather(c0_v, [sidx])
                v1_v[pl.ds(g * L, L)] = p * plsc.load_gather(c1_v, [sidx])

            pltpu.sync_copy(v0_v, n0_sh.at[dst_v], add=True)
            pltpu.sync_copy(v1_v, n1_sh.at[dst_v], add=True)
            pltpu.sync_copy(p_v, s_sh.at[dst_v], add=True)
            return 0
        lax.fori_loop(0, CHUNKS, chunk, 0)

        plsc.subcore_barrier()
        pltpu.sync_copy(n0_sh.at[pl.ds(sid * RPT, RPT)],
                        n0_out.at[cid, pl.ds(sid * RPT, RPT)])
        pltpu.sync_copy(n1_sh.at[pl.ds(sid * RPT, RPT)],
                        n1_out.at[cid, pl.ds(sid * RPT, RPT)])
        pltpu.sync_copy(s_sh.at[pl.ds(sid * RPT, RPT)],
                        s_out.at[cid, pl.ds(sid * RPT, RPT)])

    return k(srcp, dstp, as2, ad2, h2c0, h2c1, mvec)


NR = NP // 128


def _tc_c_body(n0_ref, n1_ref, s_ref, b2_ref, y_ref):
    n0 = n0_ref[0] + n0_ref[1]
    n1 = n1_ref[0] + n1_ref[1]
    s = s_ref[0] + s_ref[1]
    s = jnp.where(s > 0.0, s, 1.0)
    o0 = n0 / s + b2_ref[0, 0]
    o1 = n1 / s + b2_ref[0, 1]
    nid = (lax.broadcasted_iota(jnp.int32, (NR, 128), 0) * 128
           + lax.broadcasted_iota(jnp.int32, (NR, 128), 1))
    valid = nid < N
    mn = jnp.minimum(jnp.min(jnp.where(valid, o0, jnp.inf)),
                     jnp.min(jnp.where(valid, o1, jnp.inf)))
    mx = jnp.maximum(jnp.max(jnp.where(valid, o0, -jnp.inf)),
                     jnp.max(jnp.where(valid, o1, -jnp.inf)))
    rng = mx - mn
    y_ref[0] = 2.0 * ((o0 - mn) / rng) - 1.0
    y_ref[1] = 2.0 * ((o1 - mn) / rng) - 1.0


def _tc_c(n0_p, n1_p, s2_p, b2):
    return pl.pallas_call(
        _tc_c_body,
        in_specs=[
            pl.BlockSpec((NC, NR, 128), lambda: (0, 0, 0)),
            pl.BlockSpec((NC, NR, 128), lambda: (0, 0, 0)),
            pl.BlockSpec((NC, NR, 128), lambda: (0, 0, 0)),
            pl.BlockSpec(memory_space=pltpu.SMEM),
        ],
        out_specs=pl.BlockSpec((C, NR, 128), lambda: (0, 0, 0)),
        out_shape=jax.ShapeDtypeStruct((C, NR, 128), jnp.float32),
    )(n0_p.reshape(NC, NR, 128), n1_p.reshape(NC, NR, 128),
      s2_p.reshape(NC, NR, 128), b2.reshape(1, 2))


def kernel(x, edge_index, W1, a_src1, a_dst1, b1, W2, a_src2, a_dst2, b2):
    xp = jnp.zeros((NP, D_IN), jnp.float32).at[:N].set(x)
    loop = jnp.arange(N, dtype=jnp.int32)
    pad = jnp.arange(E_PAD - E_TOT, dtype=jnp.int32) % N
    srcp = jnp.concatenate([edge_index[0].astype(jnp.int32), loop, pad])
    dstp = jnp.concatenate([edge_index[1].astype(jnp.int32), loop, pad])

    h1, as1, ad1, m1 = _tc_a(xp, W1, a_src1, a_dst1)
    mvec1 = jnp.broadcast_to(m1.reshape(1), (L,))
    num_p, s_p = _sc1(srcp, dstp, as1.reshape(NP), ad1.reshape(NP), h1, mvec1)

    h2c0, h2c1, as2, ad2, m2 = _tc_b(num_p, s_p, b1, W2, a_src2, a_dst2)
    mvec2 = jnp.broadcast_to(m2.reshape(1), (L,))
    n0_p, n1_p, s2_p = _sc2(srcp, dstp, as2.reshape(NP), ad2.reshape(NP),
                            h2c0.reshape(NP), h2c1.reshape(NP), mvec2)

    y = _tc_c(n0_p, n1_p, s2_p, b2)
    return jnp.stack([y.reshape(C, NP)[0, :N], y.reshape(C, NP)[1, :N]], axis=1)
